# baseline (speedup 1.0000x reference)
import jax
import jax.numpy as jnp
from jax.experimental import pallas as pl
from jax.experimental.pallas import tpu as pltpu

_K = 300
_CONF = 0.1
_IOU = 0.6
_N = 20000
_NCOL = 85
_D = 516


_R = 160


def _select_nms_kernel(out_ref, boxes_ref, mask_ref, cbox_ref, sc_ref, cs_ref,
                       sup_ref):
    o = out_ref[0]
    obj = o[:, 4:5]
    clsm = jnp.max(o[:, 5:_NCOL], axis=1, keepdims=True)
    conf = obj * clsm
    conf = jnp.where(conf > _CONF, conf, 0.0)
    pad = jnp.full((_R * 128 - _N, 1), -1.0, jnp.float32)
    cs = jnp.concatenate([conf, pad], axis=0).reshape(_R, 128)
    cs_ref[:] = cs
    rowmax = jnp.max(cs, axis=1, keepdims=True).reshape(1, _R)
    i_r = jax.lax.broadcasted_iota(jnp.int32, (1, _R), 1)
    i_l = jax.lax.broadcasted_iota(jnp.int32, (1, 128), 1)

    def body(t, mrow):
        m = jnp.max(mrow)
        r = jnp.min(jnp.where(mrow == m, i_r, _R))
        row = cs_ref[pl.ds(r, 1), :]
        l = jnp.min(jnp.where(row == m, i_l, 128))
        idx = r * 128 + l
        sc_ref[pl.ds(t, 1), :] = jnp.reshape(m, (1, 1))
        cbox_ref[pl.ds(t, 1), :] = out_ref[0, pl.ds(idx, 1), 0:4]
        newrow = jnp.where(i_l == l, -1.0, row)
        cs_ref[pl.ds(r, 1), :] = newrow
        return jnp.where(i_r == r, jnp.max(newrow), mrow)

    jax.lax.fori_loop(0, _K, body, rowmax)

    cb = cbox_ref[:]
    cx = cb[:, 0:1]
    cy = cb[:, 1:2]
    w = cb[:, 2:3]
    h = cb[:, 3:4]
    x1 = cx - w * 0.5
    y1 = cy - h * 0.5
    x2 = cx + w * 0.5
    y2 = cy + h * 0.5
    boxes_ref[0] = jnp.concatenate([x1, y1, x2, y2], axis=1)

    area = (x2 - x1) * (y2 - y1)
    x1r = x1.reshape(1, _K)
    y1r = y1.reshape(1, _K)
    x2r = x2.reshape(1, _K)
    y2r = y2.reshape(1, _K)
    iw = jnp.maximum(jnp.minimum(x2, x2r) - jnp.maximum(x1, x1r), 0.0)
    ih = jnp.maximum(jnp.minimum(y2, y2r) - jnp.maximum(y1, y1r), 0.0)
    inter = iw * ih
    iou = inter / (area + area.reshape(1, _K) - inter + 1e-9)
    ri = jax.lax.broadcasted_iota(jnp.int32, (_K, _K), 0)
    ci = jax.lax.broadcasted_iota(jnp.int32, (_K, _K), 1)
    sup_ref[:] = ((iou > _IOU) & (ci > ri)).astype(jnp.float32)

    def wcond(c):
        return c[2] & (c[1] < _K + 2)

    def wbody(c):
        keep, it, _ = c
        s = jax.lax.dot_general(keep, sup_ref[:], (((1,), (0,)), ((), ())),
                                preferred_element_type=jnp.float32)
        knew = (s == 0.0).astype(jnp.float32)
        return (knew, it + 1, jnp.any(knew != keep))

    keep, _, _ = jax.lax.while_loop(
        wcond, wbody,
        (jnp.ones((1, _K), jnp.float32), jnp.int32(0), jnp.bool_(True)))
    scores = sc_ref[:].reshape(1, _K)
    mask_ref[0] = keep * (scores > 0.0).astype(jnp.float32)


def _roi_mlp_gnn_kernel(cxy_ref, boxes_ref, mask_ref, f1_ref, f2_ref, f3_ref,
                        shp_ref, w1_ref, b1_ref, w2_ref, b2_ref,
                        res_ref, cell_ref, F_ref):
    def gather(t, carry):
        cxs = cxy_ref[0, t, 0]
        cys = cxy_ref[0, t, 1]
        off = 0
        for fref, hh, ww, cc, sc in ((f1_ref, 80, 80, 192, 8.0),
                                     (f2_ref, 40, 40, 384, 16.0),
                                     (f3_ref, 20, 20, 768, 32.0)):
            xs = cxs / sc
            ys = cys / sc
            x0f = jnp.floor(xs)
            y0f = jnp.floor(ys)
            x0 = jnp.clip(x0f.astype(jnp.int32), 0, ww - 1)
            x1i = jnp.minimum(x0 + 1, ww - 1)
            y0 = jnp.clip(y0f.astype(jnp.int32), 0, hh - 1)
            y1i = jnp.minimum(y0 + 1, hh - 1)
            wx = xs - x0f
            wy = ys - y0f
            v00 = fref[0, pl.ds(y0 * ww + x0, 1), :]
            v01 = fref[0, pl.ds(y0 * ww + x1i, 1), :]
            v10 = fref[0, pl.ds(y1i * ww + x0, 1), :]
            v11 = fref[0, pl.ds(y1i * ww + x1i, 1), :]
            v = (v00 * ((1.0 - wx) * (1.0 - wy)) + v01 * (wx * (1.0 - wy))
                 + v10 * ((1.0 - wx) * wy) + v11 * (wx * wy))
            F_ref[pl.ds(t, 1), off:off + cc] = v
            off += cc
        return carry

    jax.lax.fori_loop(0, _K, gather, 0)

    F = F_ref[:]
    h = jax.lax.dot_general(F, w1_ref[:], (((1,), (0,)), ((), ())),
                            preferred_element_type=jnp.float32) + b1_ref[:]
    h = jnp.where(h > 0, h, 0.01 * h)
    h = jax.lax.dot_general(h, w2_ref[:], (((1,), (0,)), ((), ())),
                            preferred_element_type=jnp.float32) + b2_ref[:]
    h = jnp.where(h > 0, h, 0.01 * h)

    sh0 = shp_ref[0, 0, 0]
    sh1 = shp_ref[0, 0, 1]
    lane4 = jax.lax.broadcasted_iota(jnp.int32, (1, 4), 1)
    norm = jnp.where(lane4 % 2 == 0, sh1, sh0)
    bn = boxes_ref[0] / norm
    feat = jnp.concatenate([bn, h], axis=1)
    maskc = mask_ref[0].reshape(_K, 1)
    feat = jnp.where(maskc > 0, feat, 0.0)

    logits = jax.lax.dot_general(feat, feat, (((1,), (1,)), ((), ())),
                                 preferred_element_type=jnp.float32)
    logits = logits * (1.0 / jnp.sqrt(jnp.float32(_D)))
    mx = jnp.max(logits, axis=1, keepdims=True)
    e = jnp.exp(logits - mx)
    adj = e / jnp.sum(e, axis=1, keepdims=True)
    res = jax.lax.dot_general(adj, feat, (((1,), (0,)), ((), ())),
                              preferred_element_type=jnp.float32)
    res_ref[0] = res
    cell_ref[0] = jnp.mean(res, axis=0, keepdims=True)


def kernel(out, train_out, x1, x2, x3, shapes, W1, b1, W2, b2):
    B = out.shape[0]

    boxes, maskf = pl.pallas_call(
        _select_nms_kernel,
        grid=(B,),
        in_specs=[pl.BlockSpec((1, _N, _NCOL), lambda b: (b, 0, 0))],
        out_specs=[pl.BlockSpec((1, _K, 4), lambda b: (b, 0, 0)),
                   pl.BlockSpec((1, 1, _K), lambda b: (b, 0, 0))],
        out_shape=[jax.ShapeDtypeStruct((B, _K, 4), jnp.float32),
                   jax.ShapeDtypeStruct((B, 1, _K), jnp.float32)],
        scratch_shapes=[pltpu.VMEM((_K, 4), jnp.float32),
                        pltpu.VMEM((_K, 1), jnp.float32),
                        pltpu.VMEM((_R, 128), jnp.float32),
                        pltpu.VMEM((_K, _K), jnp.float32)],
    )(out)

    cxy = (boxes[:, :, 0:2] + boxes[:, :, 2:4]) * 0.5

    f1 = x1.transpose(0, 2, 3, 1).reshape(B, 6400, 192)
    f2 = x2.transpose(0, 2, 3, 1).reshape(B, 1600, 384)
    f3 = x3.transpose(0, 2, 3, 1).reshape(B, 400, 768)
    shp = shapes.reshape(B, 1, 2)
    b1r = b1.reshape(1, 512)
    b2r = b2.reshape(1, 512)

    result, cells = pl.pallas_call(
        _roi_mlp_gnn_kernel,
        grid=(B,),
        in_specs=[
            pl.BlockSpec((1, _K, 2), lambda b: (b, 0, 0),
                         memory_space=pltpu.SMEM),
            pl.BlockSpec((1, _K, 4), lambda b: (b, 0, 0)),
            pl.BlockSpec((1, 1, _K), lambda b: (b, 0, 0)),
            pl.BlockSpec((1, 6400, 192), lambda b: (b, 0, 0)),
            pl.BlockSpec((1, 1600, 384), lambda b: (b, 0, 0)),
            pl.BlockSpec((1, 400, 768), lambda b: (b, 0, 0)),
            pl.BlockSpec((1, 1, 2), lambda b: (b, 0, 0)),
            pl.BlockSpec((1344, 512), lambda b: (0, 0)),
            pl.BlockSpec((1, 512), lambda b: (0, 0)),
            pl.BlockSpec((512, 512), lambda b: (0, 0)),
            pl.BlockSpec((1, 512), lambda b: (0, 0)),
        ],
        out_specs=[pl.BlockSpec((1, _K, _D), lambda b: (b, 0, 0)),
                   pl.BlockSpec((1, 1, _D), lambda b: (b, 0, 0))],
        out_shape=[jax.ShapeDtypeStruct((B, _K, _D), jnp.float32),
                   jax.ShapeDtypeStruct((B, 1, _D), jnp.float32)],
        scratch_shapes=[pltpu.VMEM((_K, 1344), jnp.float32)],
    )(cxy, boxes, maskf, f1, f2, f3, shp, W1, b1r, W2, b2r)

    return (train_out, out, result, cells.reshape(B, _D))


# speedup vs baseline: 1.5623x; 1.5623x over previous
import jax
import jax.numpy as jnp
from jax.experimental import pallas as pl
from jax.experimental.pallas import tpu as pltpu

_K = 300
_CONF = 0.1
_IOU = 0.6
_N = 20000
_NCOL = 85
_D = 516


_R = 160


def _select_nms_kernel(out_ref, boxes_ref, mask_ref, cbox_ref, sc_ref,
                       sup_ref):
    o = out_ref[0]
    obj = o[:, 4:5]
    clsm = jnp.max(o[:, 5:_NCOL], axis=1, keepdims=True)
    conf = obj * clsm
    conf = jnp.where(conf > _CONF, conf, 0.0)
    conf2 = conf.reshape(8, _N // 8)
    iota = (jax.lax.broadcasted_iota(jnp.int32, conf2.shape, 0) * (_N // 8)
            + jax.lax.broadcasted_iota(jnp.int32, conf2.shape, 1))

    def body(t, c):
        m = jnp.max(c)
        idx = jnp.min(jnp.where(c == m, iota, _N))
        sc_ref[pl.ds(t, 1), :] = jnp.reshape(m, (1, 1))
        cbox_ref[pl.ds(t, 1), :] = out_ref[0, pl.ds(idx, 1), 0:4]
        return jnp.where(iota == idx, -1.0, c)

    jax.lax.fori_loop(0, _K, body, conf2)

    cb = cbox_ref[:]
    cx = cb[:, 0:1]
    cy = cb[:, 1:2]
    w = cb[:, 2:3]
    h = cb[:, 3:4]
    x1 = cx - w * 0.5
    y1 = cy - h * 0.5
    x2 = cx + w * 0.5
    y2 = cy + h * 0.5
    boxes_ref[0] = jnp.concatenate([x1, y1, x2, y2], axis=1)

    area = (x2 - x1) * (y2 - y1)
    x1r = x1.reshape(1, _K)
    y1r = y1.reshape(1, _K)
    x2r = x2.reshape(1, _K)
    y2r = y2.reshape(1, _K)
    iw = jnp.maximum(jnp.minimum(x2, x2r) - jnp.maximum(x1, x1r), 0.0)
    ih = jnp.maximum(jnp.minimum(y2, y2r) - jnp.maximum(y1, y1r), 0.0)
    inter = iw * ih
    iou = inter / (area + area.reshape(1, _K) - inter + 1e-9)
    ri = jax.lax.broadcasted_iota(jnp.int32, (_K, _K), 0)
    ci = jax.lax.broadcasted_iota(jnp.int32, (_K, _K), 1)
    sup_ref[:] = ((iou > _IOU) & (ci > ri)).astype(jnp.float32)

    def wcond(c):
        return c[2] & (c[1] < _K + 2)

    def wbody(c):
        keep, it, _ = c
        s = jax.lax.dot_general(keep, sup_ref[:], (((1,), (0,)), ((), ())),
                                preferred_element_type=jnp.float32)
        knew = (s == 0.0).astype(jnp.float32)
        return (knew, it + 1, jnp.any(knew != keep))

    keep, _, _ = jax.lax.while_loop(
        wcond, wbody,
        (jnp.ones((1, _K), jnp.float32), jnp.int32(0), jnp.bool_(True)))
    scores = sc_ref[:].reshape(1, _K)
    mask_ref[0] = keep * (scores > 0.0).astype(jnp.float32)


def _roi_mlp_gnn_kernel(cxy_ref, boxes_ref, mask_ref, f1_ref, f2_ref, f3_ref,
                        shp_ref, w1_ref, b1_ref, w2_ref, b2_ref,
                        res_ref, cell_ref, F_ref):
    def gather(t, carry):
        cxs = cxy_ref[0, t, 0]
        cys = cxy_ref[0, t, 1]
        off = 0
        for fref, hh, ww, cc, sc in ((f1_ref, 80, 80, 192, 8.0),
                                     (f2_ref, 40, 40, 384, 16.0),
                                     (f3_ref, 20, 20, 768, 32.0)):
            xs = cxs / sc
            ys = cys / sc
            x0f = jnp.floor(xs)
            y0f = jnp.floor(ys)
            x0 = jnp.clip(x0f.astype(jnp.int32), 0, ww - 1)
            x1i = jnp.minimum(x0 + 1, ww - 1)
            y0 = jnp.clip(y0f.astype(jnp.int32), 0, hh - 1)
            y1i = jnp.minimum(y0 + 1, hh - 1)
            wx = xs - x0f
            wy = ys - y0f
            v00 = fref[0, pl.ds(y0 * ww + x0, 1), :]
            v01 = fref[0, pl.ds(y0 * ww + x1i, 1), :]
            v10 = fref[0, pl.ds(y1i * ww + x0, 1), :]
            v11 = fref[0, pl.ds(y1i * ww + x1i, 1), :]
            v = (v00 * ((1.0 - wx) * (1.0 - wy)) + v01 * (wx * (1.0 - wy))
                 + v10 * ((1.0 - wx) * wy) + v11 * (wx * wy))
            F_ref[pl.ds(t, 1), off:off + cc] = v
            off += cc
        return carry

    jax.lax.fori_loop(0, _K, gather, 0)

    F = F_ref[:]
    h = jax.lax.dot_general(F, w1_ref[:], (((1,), (0,)), ((), ())),
                            preferred_element_type=jnp.float32) + b1_ref[:]
    h = jnp.where(h > 0, h, 0.01 * h)
    h = jax.lax.dot_general(h, w2_ref[:], (((1,), (0,)), ((), ())),
                            preferred_element_type=jnp.float32) + b2_ref[:]
    h = jnp.where(h > 0, h, 0.01 * h)

    sh0 = shp_ref[0, 0, 0]
    sh1 = shp_ref[0, 0, 1]
    lane4 = jax.lax.broadcasted_iota(jnp.int32, (1, 4), 1)
    norm = jnp.where(lane4 % 2 == 0, sh1, sh0)
    bn = boxes_ref[0] / norm
    feat = jnp.concatenate([bn, h], axis=1)
    maskc = mask_ref[0].reshape(_K, 1)
    feat = jnp.where(maskc > 0, feat, 0.0)

    logits = jax.lax.dot_general(feat, feat, (((1,), (1,)), ((), ())),
                                 preferred_element_type=jnp.float32)
    logits = logits * (1.0 / jnp.sqrt(jnp.float32(_D)))
    mx = jnp.max(logits, axis=1, keepdims=True)
    e = jnp.exp(logits - mx)
    adj = e / jnp.sum(e, axis=1, keepdims=True)
    res = jax.lax.dot_general(adj, feat, (((1,), (0,)), ((), ())),
                              preferred_element_type=jnp.float32)
    res_ref[0] = res
    cell_ref[0] = jnp.mean(res, axis=0, keepdims=True)


def kernel(out, train_out, x1, x2, x3, shapes, W1, b1, W2, b2):
    B = out.shape[0]

    boxes, maskf = pl.pallas_call(
        _select_nms_kernel,
        grid=(B,),
        in_specs=[pl.BlockSpec((1, _N, _NCOL), lambda b: (b, 0, 0))],
        out_specs=[pl.BlockSpec((1, _K, 4), lambda b: (b, 0, 0)),
                   pl.BlockSpec((1, 1, _K), lambda b: (b, 0, 0))],
        out_shape=[jax.ShapeDtypeStruct((B, _K, 4), jnp.float32),
                   jax.ShapeDtypeStruct((B, 1, _K), jnp.float32)],
        scratch_shapes=[pltpu.VMEM((_K, 4), jnp.float32),
                        pltpu.VMEM((_K, 1), jnp.float32),
                        pltpu.VMEM((_K, _K), jnp.float32)],
    )(out)

    cxy = (boxes[:, :, 0:2] + boxes[:, :, 2:4]) * 0.5

    f1 = x1.transpose(0, 2, 3, 1).reshape(B, 6400, 192)
    f2 = x2.transpose(0, 2, 3, 1).reshape(B, 1600, 384)
    f3 = x3.transpose(0, 2, 3, 1).reshape(B, 400, 768)
    shp = shapes.reshape(B, 1, 2)
    b1r = b1.reshape(1, 512)
    b2r = b2.reshape(1, 512)

    result, cells = pl.pallas_call(
        _roi_mlp_gnn_kernel,
        grid=(B,),
        in_specs=[
            pl.BlockSpec((1, _K, 2), lambda b: (b, 0, 0),
                         memory_space=pltpu.SMEM),
            pl.BlockSpec((1, _K, 4), lambda b: (b, 0, 0)),
            pl.BlockSpec((1, 1, _K), lambda b: (b, 0, 0)),
            pl.BlockSpec((1, 6400, 192), lambda b: (b, 0, 0)),
            pl.BlockSpec((1, 1600, 384), lambda b: (b, 0, 0)),
            pl.BlockSpec((1, 400, 768), lambda b: (b, 0, 0)),
            pl.BlockSpec((1, 1, 2), lambda b: (b, 0, 0)),
            pl.BlockSpec((1344, 512), lambda b: (0, 0)),
            pl.BlockSpec((1, 512), lambda b: (0, 0)),
            pl.BlockSpec((512, 512), lambda b: (0, 0)),
            pl.BlockSpec((1, 512), lambda b: (0, 0)),
        ],
        out_specs=[pl.BlockSpec((1, _K, _D), lambda b: (b, 0, 0)),
                   pl.BlockSpec((1, 1, _D), lambda b: (b, 0, 0))],
        out_shape=[jax.ShapeDtypeStruct((B, _K, _D), jnp.float32),
                   jax.ShapeDtypeStruct((B, 1, _D), jnp.float32)],
        scratch_shapes=[pltpu.VMEM((_K, 1344), jnp.float32)],
    )(cxy, boxes, maskf, f1, f2, f3, shp, W1, b1r, W2, b2r)

    return (train_out, out, result, cells.reshape(B, _D))


# speedup vs baseline: 1.5638x; 1.0009x over previous
import jax
import jax.numpy as jnp
from jax.experimental import pallas as pl
from jax.experimental.pallas import tpu as pltpu

_K = 300
_CONF = 0.1
_IOU = 0.6
_N = 20000
_NCOL = 85
_D = 516


def _fused_kernel(out_ref, f1_ref, f2_ref, f3_ref, shp_ref,
                  w1_ref, b1_ref, w2_ref, b2_ref,
                  res_ref, cell_ref, cbox_ref, sc_ref, sup_ref, F_ref):
    o = out_ref[0]
    obj = o[:, 4:5]
    clsm = jnp.max(o[:, 5:_NCOL], axis=1, keepdims=True)
    conf = obj * clsm
    conf = jnp.where(conf > _CONF, conf, 0.0)
    conf2 = conf.reshape(8, _N // 8)
    iota = (jax.lax.broadcasted_iota(jnp.int32, conf2.shape, 0) * (_N // 8)
            + jax.lax.broadcasted_iota(jnp.int32, conf2.shape, 1))

    def body(t, c):
        m = jnp.max(c)
        idx = jnp.min(jnp.where(c == m, iota, _N))
        sc_ref[pl.ds(t, 1), :] = jnp.reshape(m, (1, 1))
        cbox_ref[pl.ds(t, 1), :] = out_ref[0, pl.ds(idx, 1), 0:4]
        return jnp.where(iota == idx, -1.0, c)

    def body2(t, c):
        c = body(2 * t, c)
        return body(2 * t + 1, c)

    jax.lax.fori_loop(0, _K // 2, body2, conf2)

    cb = cbox_ref[:]
    cx = cb[:, 0:1]
    cy = cb[:, 1:2]
    w = cb[:, 2:3]
    h = cb[:, 3:4]
    x1 = cx - w * 0.5
    y1 = cy - h * 0.5
    x2 = cx + w * 0.5
    y2 = cy + h * 0.5
    boxes = jnp.concatenate([x1, y1, x2, y2], axis=1)

    area = (x2 - x1) * (y2 - y1)
    x1r = x1.reshape(1, _K)
    y1r = y1.reshape(1, _K)
    x2r = x2.reshape(1, _K)
    y2r = y2.reshape(1, _K)
    iw = jnp.maximum(jnp.minimum(x2, x2r) - jnp.maximum(x1, x1r), 0.0)
    ih = jnp.maximum(jnp.minimum(y2, y2r) - jnp.maximum(y1, y1r), 0.0)
    inter = iw * ih
    iou = inter / (area + area.reshape(1, _K) - inter + 1e-9)
    ri = jax.lax.broadcasted_iota(jnp.int32, (_K, _K), 0)
    ci = jax.lax.broadcasted_iota(jnp.int32, (_K, _K), 1)
    sup_ref[:] = ((iou > _IOU) & (ci > ri)).astype(jnp.float32)

    def wcond(c):
        return c[2] & (c[1] < _K + 2)

    def wbody(c):
        keep, it, _ = c
        s = jax.lax.dot_general(keep, sup_ref[:], (((1,), (0,)), ((), ())),
                                preferred_element_type=jnp.float32)
        knew = (s == 0.0).astype(jnp.float32)
        return (knew, it + 1, jnp.any(knew != keep))

    keep, _, _ = jax.lax.while_loop(
        wcond, wbody,
        (jnp.ones((1, _K), jnp.float32), jnp.int32(0), jnp.bool_(True)))
    scores = sc_ref[:].reshape(1, _K)
    mask = keep * (scores > 0.0).astype(jnp.float32)

    def gather(t, carry):
        cxs = cbox_ref[t, 0]
        cys = cbox_ref[t, 1]
        off = 0
        for fref, hh, ww, cc, sc in ((f1_ref, 80, 80, 192, 8.0),
                                     (f2_ref, 40, 40, 384, 16.0),
                                     (f3_ref, 20, 20, 768, 32.0)):
            xs = cxs / sc
            ys = cys / sc
            x0f = jnp.floor(xs)
            y0f = jnp.floor(ys)
            x0 = jnp.clip(x0f.astype(jnp.int32), 0, ww - 1)
            x1i = jnp.minimum(x0 + 1, ww - 1)
            y0 = jnp.clip(y0f.astype(jnp.int32), 0, hh - 1)
            y1i = jnp.minimum(y0 + 1, hh - 1)
            wx = xs - x0f
            wy = ys - y0f
            v00 = fref[0, pl.ds(y0 * ww + x0, 1), :]
            v01 = fref[0, pl.ds(y0 * ww + x1i, 1), :]
            v10 = fref[0, pl.ds(y1i * ww + x0, 1), :]
            v11 = fref[0, pl.ds(y1i * ww + x1i, 1), :]
            v = (v00 * ((1.0 - wx) * (1.0 - wy)) + v01 * (wx * (1.0 - wy))
                 + v10 * ((1.0 - wx) * wy) + v11 * (wx * wy))
            F_ref[pl.ds(t, 1), off:off + cc] = v
            off += cc
        return carry

    def gather4(t, carry):
        gather(4 * t, carry)
        gather(4 * t + 1, carry)
        gather(4 * t + 2, carry)
        gather(4 * t + 3, carry)
        return carry

    jax.lax.fori_loop(0, _K // 4, gather4, 0)

    F = F_ref[:]
    hh1 = jax.lax.dot_general(F, w1_ref[:], (((1,), (0,)), ((), ())),
                              preferred_element_type=jnp.float32) + b1_ref[:]
    hh1 = jnp.where(hh1 > 0, hh1, 0.01 * hh1)
    hh2 = jax.lax.dot_general(hh1, w2_ref[:], (((1,), (0,)), ((), ())),
                              preferred_element_type=jnp.float32) + b2_ref[:]
    hh2 = jnp.where(hh2 > 0, hh2, 0.01 * hh2)

    sh0 = shp_ref[0, 0, 0]
    sh1 = shp_ref[0, 0, 1]
    lane4 = jax.lax.broadcasted_iota(jnp.int32, (1, 4), 1)
    norm = jnp.where(lane4 % 2 == 0, sh1, sh0)
    bn = boxes / norm
    feat = jnp.concatenate([bn, hh2], axis=1)
    maskc = mask.reshape(_K, 1)
    feat = jnp.where(maskc > 0, feat, 0.0)

    logits = jax.lax.dot_general(feat, feat, (((1,), (1,)), ((), ())),
                                 preferred_element_type=jnp.float32)
    logits = logits * (1.0 / jnp.sqrt(jnp.float32(_D)))
    mx = jnp.max(logits, axis=1, keepdims=True)
    e = jnp.exp(logits - mx)
    adj = e / jnp.sum(e, axis=1, keepdims=True)
    res = jax.lax.dot_general(adj, feat, (((1,), (0,)), ((), ())),
                              preferred_element_type=jnp.float32)
    res_ref[0] = res
    cell_ref[0] = jnp.mean(res, axis=0, keepdims=True)


def kernel(out, train_out, x1, x2, x3, shapes, W1, b1, W2, b2):
    B = out.shape[0]

    f1 = x1.transpose(0, 2, 3, 1).reshape(B, 6400, 192)
    f2 = x2.transpose(0, 2, 3, 1).reshape(B, 1600, 384)
    f3 = x3.transpose(0, 2, 3, 1).reshape(B, 400, 768)
    shp = shapes.reshape(B, 1, 2)
    b1r = b1.reshape(1, 512)
    b2r = b2.reshape(1, 512)

    result, cells = pl.pallas_call(
        _fused_kernel,
        grid=(B,),
        in_specs=[
            pl.BlockSpec((1, _N, _NCOL), lambda b: (b, 0, 0)),
            pl.BlockSpec((1, 6400, 192), lambda b: (b, 0, 0)),
            pl.BlockSpec((1, 1600, 384), lambda b: (b, 0, 0)),
            pl.BlockSpec((1, 400, 768), lambda b: (b, 0, 0)),
            pl.BlockSpec((1, 1, 2), lambda b: (b, 0, 0)),
            pl.BlockSpec((1344, 512), lambda b: (0, 0)),
            pl.BlockSpec((1, 512), lambda b: (0, 0)),
            pl.BlockSpec((512, 512), lambda b: (0, 0)),
            pl.BlockSpec((1, 512), lambda b: (0, 0)),
        ],
        out_specs=[pl.BlockSpec((1, _K, _D), lambda b: (b, 0, 0)),
                   pl.BlockSpec((1, 1, _D), lambda b: (b, 0, 0))],
        out_shape=[jax.ShapeDtypeStruct((B, _K, _D), jnp.float32),
                   jax.ShapeDtypeStruct((B, 1, _D), jnp.float32)],
        scratch_shapes=[pltpu.VMEM((_K, 4), jnp.float32),
                        pltpu.VMEM((_K, 1), jnp.float32),
                        pltpu.VMEM((_K, _K), jnp.float32),
                        pltpu.VMEM((_K, 1344), jnp.float32)],
    )(out, f1, f2, f3, shp, W1, b1r, W2, b2r)

    return (train_out, out, result, cells.reshape(B, _D))


# speedup vs baseline: 1.6677x; 1.0665x over previous
import jax
import jax.numpy as jnp
from jax.experimental import pallas as pl
from jax.experimental.pallas import tpu as pltpu

_K = 300
_CONF = 0.1
_IOU = 0.6
_N = 20000
_NCOL = 85
_D = 516


_R = 160


def _select_nms_kernel(out_ref, boxes_ref, mask_ref, cbox_ref, sc_ref,
                       sup_ref):
    o = out_ref[0]
    obj = o[:, 4:5]
    clsm = jnp.max(o[:, 5:_NCOL], axis=1, keepdims=True)
    conf = obj * clsm
    conf = jnp.where(conf > _CONF, conf, 0.0)
    conf2 = conf.reshape(8, _N // 8)
    iota = (jax.lax.broadcasted_iota(jnp.int32, conf2.shape, 0) * (_N // 8)
            + jax.lax.broadcasted_iota(jnp.int32, conf2.shape, 1))

    def body(t, c):
        m = jnp.max(c)
        idx = jnp.min(jnp.where(c == m, iota, _N))
        sc_ref[pl.ds(t, 1), :] = jnp.reshape(m, (1, 1))
        cbox_ref[pl.ds(t, 1), :] = out_ref[0, pl.ds(idx, 1), 0:4]
        return jnp.where(iota == idx, -1.0, c)

    def body2(t, c):
        c = body(2 * t, c)
        return body(2 * t + 1, c)

    jax.lax.fori_loop(0, _K // 2, body2, conf2)

    cb = cbox_ref[:]
    cx = cb[:, 0:1]
    cy = cb[:, 1:2]
    w = cb[:, 2:3]
    h = cb[:, 3:4]
    x1 = cx - w * 0.5
    y1 = cy - h * 0.5
    x2 = cx + w * 0.5
    y2 = cy + h * 0.5
    boxes_ref[0] = jnp.concatenate([x1, y1, x2, y2], axis=1)

    area = (x2 - x1) * (y2 - y1)
    x1r = x1.reshape(1, _K)
    y1r = y1.reshape(1, _K)
    x2r = x2.reshape(1, _K)
    y2r = y2.reshape(1, _K)
    iw = jnp.maximum(jnp.minimum(x2, x2r) - jnp.maximum(x1, x1r), 0.0)
    ih = jnp.maximum(jnp.minimum(y2, y2r) - jnp.maximum(y1, y1r), 0.0)
    inter = iw * ih
    iou = inter / (area + area.reshape(1, _K) - inter + 1e-9)
    ri = jax.lax.broadcasted_iota(jnp.int32, (_K, _K), 0)
    ci = jax.lax.broadcasted_iota(jnp.int32, (_K, _K), 1)
    sup_ref[:] = ((iou > _IOU) & (ci > ri)).astype(jnp.float32)

    def wcond(c):
        return c[2] & (c[1] < _K + 2)

    def wbody(c):
        keep, it, _ = c
        s = jax.lax.dot_general(keep, sup_ref[:], (((1,), (0,)), ((), ())),
                                preferred_element_type=jnp.float32)
        knew = (s == 0.0).astype(jnp.float32)
        return (knew, it + 1, jnp.any(knew != keep))

    keep, _, _ = jax.lax.while_loop(
        wcond, wbody,
        (jnp.ones((1, _K), jnp.float32), jnp.int32(0), jnp.bool_(True)))
    scores = sc_ref[:].reshape(1, _K)
    mask_ref[0] = keep * (scores > 0.0).astype(jnp.float32)


def _roi_mlp_gnn_kernel(cxy_ref, boxes_ref, mask_ref, f1_ref, f2_ref, f3_ref,
                        shp_ref, w1_ref, b1_ref, w2_ref, b2_ref,
                        res_ref, cell_ref, F_ref):
    def gather(t, carry):
        cxs = cxy_ref[0, t, 0]
        cys = cxy_ref[0, t, 1]
        off = 0
        for fref, hh, ww, cc, sc in ((f1_ref, 80, 80, 192, 8.0),
                                     (f2_ref, 40, 40, 384, 16.0),
                                     (f3_ref, 20, 20, 768, 32.0)):
            xs = cxs / sc
            ys = cys / sc
            x0f = jnp.floor(xs)
            y0f = jnp.floor(ys)
            x0 = jnp.clip(x0f.astype(jnp.int32), 0, ww - 1)
            x1i = jnp.minimum(x0 + 1, ww - 1)
            y0 = jnp.clip(y0f.astype(jnp.int32), 0, hh - 1)
            y1i = jnp.minimum(y0 + 1, hh - 1)
            wx = xs - x0f
            wy = ys - y0f
            v00 = fref[0, pl.ds(y0 * ww + x0, 1), :]
            v01 = fref[0, pl.ds(y0 * ww + x1i, 1), :]
            v10 = fref[0, pl.ds(y1i * ww + x0, 1), :]
            v11 = fref[0, pl.ds(y1i * ww + x1i, 1), :]
            v = (v00 * ((1.0 - wx) * (1.0 - wy)) + v01 * (wx * (1.0 - wy))
                 + v10 * ((1.0 - wx) * wy) + v11 * (wx * wy))
            F_ref[pl.ds(t, 1), off:off + cc] = v
            off += cc
        return carry

    def gather4(t, carry):
        gather(4 * t, carry)
        gather(4 * t + 1, carry)
        gather(4 * t + 2, carry)
        gather(4 * t + 3, carry)
        return carry

    jax.lax.fori_loop(0, _K // 4, gather4, 0)

    F = F_ref[:]
    h = jax.lax.dot_general(F, w1_ref[:], (((1,), (0,)), ((), ())),
                            preferred_element_type=jnp.float32) + b1_ref[:]
    h = jnp.where(h > 0, h, 0.01 * h)
    h = jax.lax.dot_general(h, w2_ref[:], (((1,), (0,)), ((), ())),
                            preferred_element_type=jnp.float32) + b2_ref[:]
    h = jnp.where(h > 0, h, 0.01 * h)

    sh0 = shp_ref[0, 0, 0]
    sh1 = shp_ref[0, 0, 1]
    lane4 = jax.lax.broadcasted_iota(jnp.int32, (1, 4), 1)
    norm = jnp.where(lane4 % 2 == 0, sh1, sh0)
    bn = boxes_ref[0] / norm
    feat = jnp.concatenate([bn, h], axis=1)
    maskc = mask_ref[0].reshape(_K, 1)
    feat = jnp.where(maskc > 0, feat, 0.0)

    logits = jax.lax.dot_general(feat, feat, (((1,), (1,)), ((), ())),
                                 preferred_element_type=jnp.float32)
    logits = logits * (1.0 / jnp.sqrt(jnp.float32(_D)))
    mx = jnp.max(logits, axis=1, keepdims=True)
    e = jnp.exp(logits - mx)
    adj = e / jnp.sum(e, axis=1, keepdims=True)
    res = jax.lax.dot_general(adj, feat, (((1,), (0,)), ((), ())),
                              preferred_element_type=jnp.float32)
    res_ref[0] = res
    cell_ref[0] = jnp.mean(res, axis=0, keepdims=True)


def kernel(out, train_out, x1, x2, x3, shapes, W1, b1, W2, b2):
    B = out.shape[0]

    boxes, maskf = pl.pallas_call(
        _select_nms_kernel,
        grid=(B,),
        in_specs=[pl.BlockSpec((1, _N, _NCOL), lambda b: (b, 0, 0))],
        out_specs=[pl.BlockSpec((1, _K, 4), lambda b: (b, 0, 0)),
                   pl.BlockSpec((1, 1, _K), lambda b: (b, 0, 0))],
        out_shape=[jax.ShapeDtypeStruct((B, _K, 4), jnp.float32),
                   jax.ShapeDtypeStruct((B, 1, _K), jnp.float32)],
        scratch_shapes=[pltpu.VMEM((_K, 4), jnp.float32),
                        pltpu.VMEM((_K, 1), jnp.float32),
                        pltpu.VMEM((_K, _K), jnp.float32)],
    )(out)

    cxy = (boxes[:, :, 0:2] + boxes[:, :, 2:4]) * 0.5

    f1 = x1.transpose(0, 2, 3, 1).reshape(B, 6400, 192)
    f2 = x2.transpose(0, 2, 3, 1).reshape(B, 1600, 384)
    f3 = x3.transpose(0, 2, 3, 1).reshape(B, 400, 768)
    shp = shapes.reshape(B, 1, 2)
    b1r = b1.reshape(1, 512)
    b2r = b2.reshape(1, 512)

    result, cells = pl.pallas_call(
        _roi_mlp_gnn_kernel,
        grid=(B,),
        in_specs=[
            pl.BlockSpec((1, _K, 2), lambda b: (b, 0, 0),
                         memory_space=pltpu.SMEM),
            pl.BlockSpec((1, _K, 4), lambda b: (b, 0, 0)),
            pl.BlockSpec((1, 1, _K), lambda b: (b, 0, 0)),
            pl.BlockSpec((1, 6400, 192), lambda b: (b, 0, 0)),
            pl.BlockSpec((1, 1600, 384), lambda b: (b, 0, 0)),
            pl.BlockSpec((1, 400, 768), lambda b: (b, 0, 0)),
            pl.BlockSpec((1, 1, 2), lambda b: (b, 0, 0)),
            pl.BlockSpec((1344, 512), lambda b: (0, 0)),
            pl.BlockSpec((1, 512), lambda b: (0, 0)),
            pl.BlockSpec((512, 512), lambda b: (0, 0)),
            pl.BlockSpec((1, 512), lambda b: (0, 0)),
        ],
        out_specs=[pl.BlockSpec((1, _K, _D), lambda b: (b, 0, 0)),
                   pl.BlockSpec((1, 1, _D), lambda b: (b, 0, 0))],
        out_shape=[jax.ShapeDtypeStruct((B, _K, _D), jnp.float32),
                   jax.ShapeDtypeStruct((B, 1, _D), jnp.float32)],
        scratch_shapes=[pltpu.VMEM((_K, 1344), jnp.float32)],
    )(cxy, boxes, maskf, f1, f2, f3, shp, W1, b1r, W2, b2r)

    return (train_out, out, result, cells.reshape(B, _D))
